# baseline (speedup 1.0000x reference)
import jax
import jax.numpy as jnp
from jax import lax
from jax.experimental import pallas as pl
from jax.experimental.pallas import tpu as pltpu
from jax.experimental.pallas import tpu_sc as plsc

N = 10000
E = 320000
D = 200
H = 5
F = 10
HF = H * F
G = 64
ROW = 64
TXW = 64
BB = 200
NBLK = 50
NSC = 2
NSUB = 16
NW = NSC * NSUB
EPW = E // NW
RPT = 640
RPT_LAST = N - RPT * (NSUB - 1)


def _prep_body(x_ref, w_ref, as_ref, ad_ref, tx_ref, tb_ref, sl_ref):
    xw = jnp.dot(x_ref[...], w_ref[...], preferred_element_type=jnp.float32)
    a_s = []
    a_d = []
    for h in range(H):
        xh = xw[:, h * F:(h + 1) * F]
        a_s.append(jnp.sum(xh * as_ref[h:h + 1, :], axis=1, keepdims=True))
        a_d.append(jnp.sum(xh * ad_ref[h:h + 1, :], axis=1, keepdims=True))
    a_s = jnp.concatenate(a_s, axis=1)
    a_d = jnp.concatenate(a_d, axis=1)
    al = a_s + a_d
    w_self = jnp.exp(jnp.maximum(al, 0.2 * al))
    sl50 = []
    for h in range(H):
        sl50.append(w_self[:, h:h + 1] * xw[:, h * F:(h + 1) * F])
    sl50 = jnp.concatenate(sl50, axis=1)
    n = xw.shape[0]
    ones5 = jnp.ones((n, H), jnp.float32)
    z9 = jnp.zeros((n, 9), jnp.float32)
    z11 = jnp.zeros((n, 11), jnp.float32)
    z4 = jnp.zeros((n, 4), jnp.float32)
    z7 = jnp.zeros((n, 7), jnp.float32)
    tx_ref[...] = jnp.concatenate([xw, ones5, a_s, z4], axis=1)
    sl_ref[...] = 0.5 * jnp.concatenate([sl50, w_self, z9], axis=1)
    tb_ref[...] = jnp.concatenate([z7, a_d, z4], axis=1)


def _prep(x, W, att_src, att_dst):
    NB = 2000
    return pl.pallas_call(
        _prep_body,
        grid=(N // NB,),
        in_specs=[
            pl.BlockSpec((NB, D), lambda i: (i, 0)),
            pl.BlockSpec((D, HF), lambda i: (0, 0)),
            pl.BlockSpec((H, F), lambda i: (0, 0)),
            pl.BlockSpec((H, F), lambda i: (0, 0)),
        ],
        out_specs=(
            pl.BlockSpec((NB, TXW), lambda i: (i, 0)),
            pl.BlockSpec((NB, 16), lambda i: (i, 0)),
            pl.BlockSpec((NB, ROW), lambda i: (i, 0)),
        ),
        out_shape=(
            jax.ShapeDtypeStruct((N, TXW), jnp.float32),
            jax.ShapeDtypeStruct((N, 16), jnp.float32),
            jax.ShapeDtypeStruct((N, ROW), jnp.float32),
        ),
    )(x, W, att_src, att_dst)


def _edge_body(tx_hbm, tb_hbm, src_hbm, dst_hbm, zero_hbm, acc_hbm,
               xr0, xr1, tb0, tb1, mm0, mm1, si_all, di_all,
               dc0, dc1, acc, sx0, sx1, sb0, sb1, sm0, sm1):
    c = lax.axis_index("c")
    s = lax.axis_index("s")
    wid = s * NSC + c
    lane = lax.iota(jnp.int32, 16)

    r0 = pl.multiple_of(s * RPT, 8)

    @pl.when(s < NSUB - 1)
    def _():
        pltpu.sync_copy(zero_hbm.at[pl.ds(r0, RPT)], acc.at[pl.ds(r0, RPT)])

    @pl.when(s == NSUB - 1)
    def _():
        pltpu.sync_copy(zero_hbm.at[pl.ds(r0, RPT_LAST)],
                        acc.at[pl.ds(r0, RPT_LAST)])

    plsc.subcore_barrier()

    hmap = []
    for j in range(4):
        k = lane + 16 * j
        q10 = (k * 205) >> 11
        q50 = (k * 41) >> 11
        hmap.append(jnp.minimum(7 + q10 - q50 * (55 - k), 15))

    base = wid * EPW
    bufs = ((xr0, tb0, mm0, dc0, sx0, sb0, sm0),
            (xr1, tb1, mm1, dc1, sx1, sb1, sm1))

    pltpu.sync_copy(src_hbm.at[pl.ds(base, EPW)], si_all)
    pltpu.sync_copy(dst_hbm.at[pl.ds(base, EPW)], di_all)

    def start_rows(kidx, xr_, tb_, sx_, sb_):
        pltpu.async_copy(tx_hbm.at[si_all.at[pl.ds(kidx * BB, BB)]], xr_, sx_)
        pltpu.async_copy(tb_hbm.at[di_all.at[pl.ds(kidx * BB, BB)]], tb_, sb_)

    start_rows(0, xr0, tb0, sx0, sb0)

    def blockpair(g, _):
        for b in (0, 1):
            xr_, tb_, mm_, dc_, sx_, sb_, sm_ = bufs[b]
            nxt = bufs[1 - b]
            k = g * 2 + b
            pltpu.make_async_copy(tx_hbm.at[si_all.at[pl.ds(0, BB)]], xr_,
                                  sx_).wait()
            pltpu.make_async_copy(tb_hbm.at[di_all.at[pl.ds(0, BB)]], tb_,
                                  sb_).wait()

            @pl.when(k >= 2)
            def _():
                pltpu.make_async_copy(mm_, acc.at[dc_], sm_).wait()

            @pl.when(k + 1 < NBLK)
            def _():
                start_rows(k + 1, nxt[0], nxt[1], nxt[4], nxt[5])

            @plsc.parallel_loop(0, BB, unroll=8)
            def _(e):
                x3 = xr_[e, pl.ds(48, 16)]
                a = x3 + tb_[e, :]
                w = jnp.exp(jnp.maximum(a, 0.2 * a))
                for j in range(4):
                    xc = x3 if j == 3 else xr_[e, pl.ds(16 * j, 16)]
                    wg = w.at[hmap[j]].get(mode="promise_in_bounds")
                    mm_[e, pl.ds(16 * j, 16)] = xc * wg

            pltpu.sync_copy(dst_hbm.at[pl.ds(base + k * BB, BB)], dc_)
            pltpu.async_copy(mm_, acc.at[dc_], sm_, add=True)
        return 0

    lax.fori_loop(0, NBLK // 2, blockpair, 0)

    pltpu.make_async_copy(mm0, acc.at[dc0], sm0).wait()
    pltpu.make_async_copy(mm1, acc.at[dc1], sm1).wait()

    plsc.subcore_barrier()

    @pl.when(s < NSUB - 1)
    def _():
        pltpu.sync_copy(acc.at[pl.ds(r0, RPT)], acc_hbm.at[c, pl.ds(r0, RPT)])

    @pl.when(s == NSUB - 1)
    def _():
        pltpu.sync_copy(acc.at[pl.ds(r0, RPT_LAST)],
                        acc_hbm.at[c, pl.ds(r0, RPT_LAST)])


def _edge(TX, TB, src, dst, zeros):
    mesh = plsc.VectorSubcoreMesh(core_axis_name="c", subcore_axis_name="s")
    f = pl.kernel(
        _edge_body,
        out_type=jax.ShapeDtypeStruct((NSC, N, ROW), jnp.float32),
        mesh=mesh,
        scratch_types=[
            pltpu.VMEM((BB, TXW), jnp.float32),
            pltpu.VMEM((BB, TXW), jnp.float32),
            pltpu.VMEM((BB, 16), jnp.float32),
            pltpu.VMEM((BB, 16), jnp.float32),
            pltpu.VMEM((BB, ROW), jnp.float32),
            pltpu.VMEM((BB, ROW), jnp.float32),
            pltpu.VMEM((EPW,), jnp.int32),
            pltpu.VMEM((EPW,), jnp.int32),
            pltpu.VMEM((BB,), jnp.int32),
            pltpu.VMEM((BB,), jnp.int32),
            pltpu.VMEM_SHARED((N, ROW), jnp.float32),
            pltpu.SemaphoreType.DMA,
            pltpu.SemaphoreType.DMA,
            pltpu.SemaphoreType.DMA,
            pltpu.SemaphoreType.DMA,
            pltpu.SemaphoreType.DMA,
            pltpu.SemaphoreType.DMA,
        ],
        compiler_params=pltpu.CompilerParams(use_tc_tiling_on_sc=False),
    )
    return f(TX, TB, src, dst, zeros)


def _post_body(acc_ref, b_ref, bias_ref, lw_ref, lb_ref, h_ref, y_ref):
    A = acc_ref[0] + acc_ref[1]
    outs = []
    for h in range(H):
        den = A[:, HF + h:HF + h + 1] + 1e-16
        outs.append(A[:, h * F:(h + 1) * F] / den)
    out = jnp.concatenate(outs, axis=1) + bias_ref[...]
    out = jnp.where(out > 0, out, jnp.exp(jnp.minimum(out, 0.0)) - 1.0)
    gid = lax.broadcasted_iota(jnp.int32, (1, G), 1)
    P = (b_ref[...] == gid).astype(jnp.float32)
    sums = lax.dot_general(P, out, (((0,), (0,)), ((), ())),
                           preferred_element_type=jnp.float32)
    cnt = lax.dot_general(P, jnp.ones((N, 1), jnp.float32),
                          (((0,), (0,)), ((), ())),
                          preferred_element_type=jnp.float32)
    hm = sums / jnp.maximum(cnt, 1.0)
    h_ref[...] = hm
    y_ref[...] = jax.nn.sigmoid(
        jnp.dot(hm, lw_ref[...], preferred_element_type=jnp.float32)
        + lb_ref[...])


def _post(ACC, batch2d, bias2d, lin_w, lin_b2d):
    return pl.pallas_call(
        _post_body,
        out_shape=(
            jax.ShapeDtypeStruct((G, HF), jnp.float32),
            jax.ShapeDtypeStruct((G, 1), jnp.float32),
        ),
    )(ACC, batch2d, bias2d, lin_w, lin_b2d)


def kernel(x, edge_index, batch, W, att_src, att_dst, bias, lin_w, lin_b):
    TX, TB, SLH = _prep(x, W, att_src, att_dst)
    ACC = _edge(TX, TB, edge_index[0], edge_index[1], SLH)
    h, y = _post(ACC, batch.reshape(N, 1), bias.reshape(1, HF),
                 lin_w, lin_b.reshape(1, 1))
    return (h, y)


# speedup vs baseline: 1.2008x; 1.2008x over previous
import jax
import jax.numpy as jnp
from jax import lax
from jax.experimental import pallas as pl
from jax.experimental.pallas import tpu as pltpu
from jax.experimental.pallas import tpu_sc as plsc

N = 10000
E = 320000
D = 200
H = 5
F = 10
HF = H * F
G = 64
ROW = 64
TXW = 64
BB = 200
NBLK = 50
NSC = 2
NSUB = 16
NW = NSC * NSUB
EPW = E // NW
RPT = 640
RPT_LAST = N - RPT * (NSUB - 1)


def _prep_body(x_ref, w_ref, as_ref, ad_ref, e5_ref, tx_ref, tb_ref, sl_ref):
    xw = jnp.dot(x_ref[...], w_ref[...], preferred_element_type=jnp.float32)
    a_s = jnp.dot(xw, as_ref[...], preferred_element_type=jnp.float32)
    a_d = jnp.dot(xw, ad_ref[...], preferred_element_type=jnp.float32)
    al = a_s + a_d
    w_self = jnp.exp(jnp.maximum(al, 0.2 * al))
    w_exp = jnp.dot(w_self, e5_ref[...], preferred_element_type=jnp.float32)
    sl50 = w_exp * xw
    n = xw.shape[0]
    ones5 = jnp.ones((n, H), jnp.float32)
    z9 = jnp.zeros((n, 9), jnp.float32)
    z11 = jnp.zeros((n, 11), jnp.float32)
    z4 = jnp.zeros((n, 4), jnp.float32)
    z7 = jnp.zeros((n, 7), jnp.float32)
    tx_ref[...] = jnp.concatenate([xw, ones5, a_s, z4], axis=1)
    sl_ref[...] = 0.5 * jnp.concatenate([sl50, w_self, z9], axis=1)
    tb_ref[...] = jnp.concatenate([z7, a_d, z4], axis=1)


def _prep(x, W, AS, AD, E5):
    NB = 2000
    return pl.pallas_call(
        _prep_body,
        grid=(N // NB,),
        in_specs=[
            pl.BlockSpec((NB, D), lambda i: (i, 0)),
            pl.BlockSpec((D, HF), lambda i: (0, 0)),
            pl.BlockSpec((HF, H), lambda i: (0, 0)),
            pl.BlockSpec((HF, H), lambda i: (0, 0)),
            pl.BlockSpec((H, HF), lambda i: (0, 0)),
        ],
        out_specs=(
            pl.BlockSpec((NB, TXW), lambda i: (i, 0)),
            pl.BlockSpec((NB, 16), lambda i: (i, 0)),
            pl.BlockSpec((NB, ROW), lambda i: (i, 0)),
        ),
        out_shape=(
            jax.ShapeDtypeStruct((N, TXW), jnp.float32),
            jax.ShapeDtypeStruct((N, 16), jnp.float32),
            jax.ShapeDtypeStruct((N, ROW), jnp.float32),
        ),
    )(x, W, AS, AD, E5)


def _edge_body(tx_hbm, tb_hbm, src_hbm, dst_hbm, zero_hbm, acc_hbm,
               xr0, xr1, tb0, tb1, mm0, mm1, si_all, di_all,
               dc0, dc1, acc, sx0, sx1, sb0, sb1, sm0, sm1):
    c = lax.axis_index("c")
    s = lax.axis_index("s")
    wid = s * NSC + c
    lane = lax.iota(jnp.int32, 16)

    r0 = pl.multiple_of(s * RPT, 8)

    @pl.when(s < NSUB - 1)
    def _():
        pltpu.sync_copy(zero_hbm.at[pl.ds(r0, RPT)], acc.at[pl.ds(r0, RPT)])

    @pl.when(s == NSUB - 1)
    def _():
        pltpu.sync_copy(zero_hbm.at[pl.ds(r0, RPT_LAST)],
                        acc.at[pl.ds(r0, RPT_LAST)])

    plsc.subcore_barrier()

    hmap = []
    for j in range(4):
        k = lane + 16 * j
        q10 = (k * 205) >> 11
        q50 = (k * 41) >> 11
        hmap.append(jnp.minimum(7 + q10 - q50 * (55 - k), 15))

    base = wid * EPW
    bufs = ((xr0, tb0, mm0, dc0, sx0, sb0, sm0),
            (xr1, tb1, mm1, dc1, sx1, sb1, sm1))

    pltpu.sync_copy(src_hbm.at[pl.ds(base, EPW)], si_all)
    pltpu.sync_copy(dst_hbm.at[pl.ds(base, EPW)], di_all)

    def start_rows(kidx, xr_, tb_, sx_, sb_):
        pltpu.async_copy(tx_hbm.at[si_all.at[pl.ds(kidx * BB, BB)]], xr_, sx_)
        pltpu.async_copy(tb_hbm.at[di_all.at[pl.ds(kidx * BB, BB)]], tb_, sb_)

    start_rows(0, xr0, tb0, sx0, sb0)

    def blockpair(g, _):
        for b in (0, 1):
            xr_, tb_, mm_, dc_, sx_, sb_, sm_ = bufs[b]
            nxt = bufs[1 - b]
            k = g * 2 + b
            pltpu.make_async_copy(tx_hbm.at[si_all.at[pl.ds(0, BB)]], xr_,
                                  sx_).wait()
            pltpu.make_async_copy(tb_hbm.at[di_all.at[pl.ds(0, BB)]], tb_,
                                  sb_).wait()

            @pl.when(k >= 2)
            def _():
                pltpu.make_async_copy(mm_, acc.at[dc_], sm_).wait()

            @pl.when(k + 1 < NBLK)
            def _():
                start_rows(k + 1, nxt[0], nxt[1], nxt[4], nxt[5])

            @plsc.parallel_loop(0, BB, unroll=8)
            def _(e):
                x3 = xr_[e, pl.ds(48, 16)]
                a = x3 + tb_[e, :]
                w = jnp.exp(jnp.maximum(a, 0.2 * a))
                for j in range(4):
                    xc = x3 if j == 3 else xr_[e, pl.ds(16 * j, 16)]
                    wg = w.at[hmap[j]].get(mode="promise_in_bounds")
                    mm_[e, pl.ds(16 * j, 16)] = xc * wg

            pltpu.sync_copy(dst_hbm.at[pl.ds(base + k * BB, BB)], dc_)
            pltpu.async_copy(mm_, acc.at[dc_], sm_, add=True)
        return 0

    lax.fori_loop(0, NBLK // 2, blockpair, 0)

    pltpu.make_async_copy(mm0, acc.at[dc0], sm0).wait()
    pltpu.make_async_copy(mm1, acc.at[dc1], sm1).wait()

    plsc.subcore_barrier()

    @pl.when(s < NSUB - 1)
    def _():
        pltpu.sync_copy(acc.at[pl.ds(r0, RPT)], acc_hbm.at[c, pl.ds(r0, RPT)])

    @pl.when(s == NSUB - 1)
    def _():
        pltpu.sync_copy(acc.at[pl.ds(r0, RPT_LAST)],
                        acc_hbm.at[c, pl.ds(r0, RPT_LAST)])


def _edge(TX, TB, src, dst, zeros):
    mesh = plsc.VectorSubcoreMesh(core_axis_name="c", subcore_axis_name="s")
    f = pl.kernel(
        _edge_body,
        out_type=jax.ShapeDtypeStruct((NSC, N, ROW), jnp.float32),
        mesh=mesh,
        scratch_types=[
            pltpu.VMEM((BB, TXW), jnp.float32),
            pltpu.VMEM((BB, TXW), jnp.float32),
            pltpu.VMEM((BB, 16), jnp.float32),
            pltpu.VMEM((BB, 16), jnp.float32),
            pltpu.VMEM((BB, ROW), jnp.float32),
            pltpu.VMEM((BB, ROW), jnp.float32),
            pltpu.VMEM((EPW,), jnp.int32),
            pltpu.VMEM((EPW,), jnp.int32),
            pltpu.VMEM((BB,), jnp.int32),
            pltpu.VMEM((BB,), jnp.int32),
            pltpu.VMEM_SHARED((N, ROW), jnp.float32),
            pltpu.SemaphoreType.DMA,
            pltpu.SemaphoreType.DMA,
            pltpu.SemaphoreType.DMA,
            pltpu.SemaphoreType.DMA,
            pltpu.SemaphoreType.DMA,
            pltpu.SemaphoreType.DMA,
        ],
        compiler_params=pltpu.CompilerParams(use_tc_tiling_on_sc=False),
    )
    return f(TX, TB, src, dst, zeros)


def _post_body(acc_ref, e5_ref, b_ref, bias_ref, lw_ref, lb_ref, h_ref, y_ref):
    A = acc_ref[0] + acc_ref[1]
    den = jnp.dot(A[:, HF:HF + H], e5_ref[...],
                  preferred_element_type=jnp.float32) + 1e-16
    out = A[:, :HF] / den + bias_ref[...]
    out = jnp.where(out > 0, out, jnp.exp(jnp.minimum(out, 0.0)) - 1.0)
    gid = lax.broadcasted_iota(jnp.int32, (1, G), 1)
    P = (b_ref[...] == gid).astype(jnp.float32)
    sums = lax.dot_general(P, out, (((0,), (0,)), ((), ())),
                           preferred_element_type=jnp.float32)
    cnt = lax.dot_general(P, jnp.ones((N, 1), jnp.float32),
                          (((0,), (0,)), ((), ())),
                          preferred_element_type=jnp.float32)
    hm = sums / jnp.maximum(cnt, 1.0)
    h_ref[...] = hm
    y_ref[...] = jax.nn.sigmoid(
        jnp.dot(hm, lw_ref[...], preferred_element_type=jnp.float32)
        + lb_ref[...])


def _post(ACC, E5, batch2d, bias2d, lin_w, lin_b2d):
    return pl.pallas_call(
        _post_body,
        out_shape=(
            jax.ShapeDtypeStruct((G, HF), jnp.float32),
            jax.ShapeDtypeStruct((G, 1), jnp.float32),
        ),
    )(ACC, E5, batch2d, bias2d, lin_w, lin_b2d)


def kernel(x, edge_index, batch, W, att_src, att_dst, bias, lin_w, lin_b):
    hsel = (jnp.arange(HF)[:, None] // F == jnp.arange(H)[None, :])
    AS = jnp.where(hsel, att_src.reshape(HF)[:, None], 0.0)
    AD = jnp.where(hsel, att_dst.reshape(HF)[:, None], 0.0)
    E5 = hsel.T.astype(jnp.float32)
    TX, TB, SLH = _prep(x, W, AS, AD, E5)
    ACC = _edge(TX, TB, edge_index[0], edge_index[1], SLH)
    h, y = _post(ACC, E5, batch.reshape(N, 1), bias.reshape(1, HF),
                 lin_w, lin_b.reshape(1, 1))
    return (h, y)


# speedup vs baseline: 1.2010x; 1.0002x over previous
import jax
import jax.numpy as jnp
from jax import lax
from jax.experimental import pallas as pl
from jax.experimental.pallas import tpu as pltpu
from jax.experimental.pallas import tpu_sc as plsc

N = 10000
E = 320000
D = 200
H = 5
F = 10
HF = H * F
G = 64
ROW = 64
TXW = 64
BB = 200
NBLK = 50
NSC = 2
NSUB = 16
NW = NSC * NSUB
EPW = E // NW
RPT = 640
RPT_LAST = N - RPT * (NSUB - 1)


def _prep_body(x_ref, w_ref, as_ref, ad_ref, e5_ref, tx_ref, tb_ref, sl_ref):
    xw = jnp.dot(x_ref[...], w_ref[...], preferred_element_type=jnp.float32)
    a_s = jnp.dot(xw, as_ref[...], preferred_element_type=jnp.float32)
    a_d = jnp.dot(xw, ad_ref[...], preferred_element_type=jnp.float32)
    al = a_s + a_d
    w_self = jnp.exp(jnp.maximum(al, 0.2 * al))
    w_exp = jnp.dot(w_self, e5_ref[...], preferred_element_type=jnp.float32)
    sl50 = w_exp * xw
    n = xw.shape[0]
    ones5 = jnp.ones((n, H), jnp.float32)
    z9 = jnp.zeros((n, 9), jnp.float32)
    z4 = jnp.zeros((n, 4), jnp.float32)
    z7 = jnp.zeros((n, 7), jnp.float32)
    tx_ref[...] = jnp.concatenate([xw, ones5, a_s, z4], axis=1)
    sl_ref[...] = 0.5 * jnp.concatenate([sl50, w_self, z9], axis=1)
    tb_ref[...] = jnp.concatenate([z7, a_d, z4], axis=1)


def _prep(x, W, AS, AD, E5):
    NB = 2000
    return pl.pallas_call(
        _prep_body,
        grid=(N // NB,),
        in_specs=[
            pl.BlockSpec((NB, D), lambda i: (i, 0)),
            pl.BlockSpec((D, HF), lambda i: (0, 0)),
            pl.BlockSpec((HF, H), lambda i: (0, 0)),
            pl.BlockSpec((HF, H), lambda i: (0, 0)),
            pl.BlockSpec((H, HF), lambda i: (0, 0)),
        ],
        out_specs=(
            pl.BlockSpec((NB, TXW), lambda i: (i, 0)),
            pl.BlockSpec((NB, 16), lambda i: (i, 0)),
            pl.BlockSpec((NB, ROW), lambda i: (i, 0)),
        ),
        out_shape=(
            jax.ShapeDtypeStruct((N, TXW), jnp.float32),
            jax.ShapeDtypeStruct((N, 16), jnp.float32),
            jax.ShapeDtypeStruct((N, ROW), jnp.float32),
        ),
    )(x, W, AS, AD, E5)


def _edge_body(tx_hbm, tb_hbm, src_hbm, dst_hbm, zero_hbm, acc_hbm,
               xr0, xr1, tb0, tb1, mm0, mm1, si_all, di_all,
               dc0, dc1, acc, sx0, sx1, sb0, sb1, sm0, sm1):
    c = lax.axis_index("c")
    s = lax.axis_index("s")
    wid = s * NSC + c
    lane = lax.iota(jnp.int32, 16)

    r0 = pl.multiple_of(s * RPT, 8)

    @pl.when(s < NSUB - 1)
    def _():
        pltpu.sync_copy(zero_hbm.at[pl.ds(r0, RPT)], acc.at[pl.ds(r0, RPT)])

    @pl.when(s == NSUB - 1)
    def _():
        pltpu.sync_copy(zero_hbm.at[pl.ds(r0, RPT_LAST)],
                        acc.at[pl.ds(r0, RPT_LAST)])

    plsc.subcore_barrier()

    hmap = []
    for j in range(4):
        k = lane + 16 * j
        q10 = (k * 205) >> 11
        q50 = (k * 41) >> 11
        hmap.append(jnp.minimum(7 + q10 - q50 * (55 - k), 15))

    base = wid * EPW
    bufs = ((xr0, tb0, mm0, dc0, sx0, sb0, sm0),
            (xr1, tb1, mm1, dc1, sx1, sb1, sm1))

    pltpu.sync_copy(src_hbm.at[pl.ds(base, EPW)], si_all)
    pltpu.sync_copy(dst_hbm.at[pl.ds(base, EPW)], di_all)

    def start_rows(kidx, xr_, tb_, sx_, sb_):
        pltpu.async_copy(tx_hbm.at[si_all.at[pl.ds(kidx * BB, BB)]], xr_, sx_)
        pltpu.async_copy(tb_hbm.at[di_all.at[pl.ds(kidx * BB, BB)]], tb_, sb_)

    start_rows(0, xr0, tb0, sx0, sb0)

    def blockpair(g, _):
        for b in (0, 1):
            xr_, tb_, mm_, dc_, sx_, sb_, sm_ = bufs[b]
            nxt = bufs[1 - b]
            k = g * 2 + b
            pltpu.make_async_copy(tx_hbm.at[si_all.at[pl.ds(0, BB)]], xr_,
                                  sx_).wait()
            pltpu.make_async_copy(tb_hbm.at[di_all.at[pl.ds(0, BB)]], tb_,
                                  sb_).wait()

            @pl.when(k >= 2)
            def _():
                pltpu.make_async_copy(mm_, acc.at[dc_], sm_).wait()

            @pl.when(k + 1 < NBLK)
            def _():
                start_rows(k + 1, nxt[0], nxt[1], nxt[4], nxt[5])

            @plsc.parallel_loop(0, BB, unroll=8)
            def _(e):
                x3 = xr_[e, pl.ds(48, 16)]
                a = x3 + tb_[e, :]
                w = jnp.exp(jnp.maximum(a, 0.2 * a))
                for j in range(4):
                    xc = x3 if j == 3 else xr_[e, pl.ds(16 * j, 16)]
                    wg = w.at[hmap[j]].get(mode="promise_in_bounds")
                    mm_[e, pl.ds(16 * j, 16)] = xc * wg

            pltpu.sync_copy(dst_hbm.at[pl.ds(base + k * BB, BB)], dc_)
            pltpu.async_copy(mm_, acc.at[dc_], sm_, add=True)
        return 0

    lax.fori_loop(0, NBLK // 2, blockpair, 0)

    pltpu.make_async_copy(mm0, acc.at[dc0], sm0).wait()
    pltpu.make_async_copy(mm1, acc.at[dc1], sm1).wait()

    plsc.subcore_barrier()

    @pl.when(s < NSUB - 1)
    def _():
        pltpu.sync_copy(acc.at[pl.ds(r0, RPT)], acc_hbm.at[c, pl.ds(r0, RPT)])

    @pl.when(s == NSUB - 1)
    def _():
        pltpu.sync_copy(acc.at[pl.ds(r0, RPT_LAST)],
                        acc_hbm.at[c, pl.ds(r0, RPT_LAST)])


def _edge(TX, TB, src, dst, zeros):
    mesh = plsc.VectorSubcoreMesh(core_axis_name="c", subcore_axis_name="s")
    f = pl.kernel(
        _edge_body,
        out_type=jax.ShapeDtypeStruct((NSC, N, ROW), jnp.float32),
        mesh=mesh,
        scratch_types=[
            pltpu.VMEM((BB, TXW), jnp.float32),
            pltpu.VMEM((BB, TXW), jnp.float32),
            pltpu.VMEM((BB, 16), jnp.float32),
            pltpu.VMEM((BB, 16), jnp.float32),
            pltpu.VMEM((BB, ROW), jnp.float32),
            pltpu.VMEM((BB, ROW), jnp.float32),
            pltpu.VMEM((EPW,), jnp.int32),
            pltpu.VMEM((EPW,), jnp.int32),
            pltpu.VMEM((BB,), jnp.int32),
            pltpu.VMEM((BB,), jnp.int32),
            pltpu.VMEM_SHARED((N, ROW), jnp.float32),
            pltpu.SemaphoreType.DMA,
            pltpu.SemaphoreType.DMA,
            pltpu.SemaphoreType.DMA,
            pltpu.SemaphoreType.DMA,
            pltpu.SemaphoreType.DMA,
            pltpu.SemaphoreType.DMA,
        ],
        compiler_params=pltpu.CompilerParams(use_tc_tiling_on_sc=False),
    )
    return f(TX, TB, src, dst, zeros)


def _post_body(acc_ref, e5_ref, b_ref, bias_ref, lw_ref, lb_ref, h_ref, y_ref):
    A = acc_ref[0] + acc_ref[1]
    den = jnp.dot(A[:, HF:HF + H], e5_ref[...],
                  preferred_element_type=jnp.float32) + 1e-16
    out = A[:, :HF] / den + bias_ref[...]
    out = jnp.where(out > 0, out, jnp.exp(jnp.minimum(out, 0.0)) - 1.0)
    gid = lax.broadcasted_iota(jnp.int32, (1, G), 1)
    P = (b_ref[...] == gid).astype(jnp.float32)
    sums = lax.dot_general(P, out, (((0,), (0,)), ((), ())),
                           preferred_element_type=jnp.float32)
    cnt = lax.dot_general(P, jnp.ones((N, 1), jnp.float32),
                          (((0,), (0,)), ((), ())),
                          preferred_element_type=jnp.float32)
    hm = sums / jnp.maximum(cnt, 1.0)
    h_ref[...] = hm
    y_ref[...] = jax.nn.sigmoid(
        jnp.dot(hm, lw_ref[...], preferred_element_type=jnp.float32)
        + lb_ref[...])


def _post(ACC, E5, batch2d, bias2d, lin_w, lin_b2d):
    return pl.pallas_call(
        _post_body,
        out_shape=(
            jax.ShapeDtypeStruct((G, HF), jnp.float32),
            jax.ShapeDtypeStruct((G, 1), jnp.float32),
        ),
    )(ACC, E5, batch2d, bias2d, lin_w, lin_b2d)


def kernel(x, edge_index, batch, W, att_src, att_dst, bias, lin_w, lin_b):
    hsel = (jnp.arange(HF)[:, None] // F == jnp.arange(H)[None, :])
    AS = jnp.where(hsel, att_src.reshape(HF)[:, None], 0.0)
    AD = jnp.where(hsel, att_dst.reshape(HF)[:, None], 0.0)
    E5 = hsel.T.astype(jnp.float32)
    TX, TB, SLH = _prep(x, W, AS, AD, E5)
    ACC = _edge(TX, TB, edge_index[0], edge_index[1], SLH)
    h, y = _post(ACC, E5, batch.reshape(N, 1), bias.reshape(1, HF),
                 lin_w, lin_b.reshape(1, 1))
    return (h, y)
